# baseline (speedup 1.0000x reference)
import functools

import jax
import jax.numpy as jnp
from jax import lax
from jax.experimental import pallas as pl
from jax.experimental.pallas import tpu as pltpu
from jax.experimental.pallas import tpu_sc as plsc

_NC, _NS = 2, 16
_NW = _NC * _NS
_B, _S, _D = 4, 4096, 1024
_SEQ_PER_W = _S // _NW
_CS = 32
_NJ = _SEQ_PER_W // _CS
_L = 16
_SCALE = 32.0


def _body(x_hbm, table_hbm, pe_hbm, out_hbm, idx_v, pe_v, rows_v, gsem):
    wid = lax.axis_index("s") * _NC + lax.axis_index("c")
    s_base = wid * _SEQ_PER_W
    pltpu.sync_copy(x_hbm.at[:, pl.ds(s_base, _SEQ_PER_W)], idx_v)
    for j in range(_NJ):
        s0 = s_base + j * _CS
        pltpu.sync_copy(pe_hbm.at[pl.ds(s0, _CS)], pe_v)
        for b in range(_B):
            pltpu.async_copy(
                table_hbm.at[idx_v.at[b, pl.ds(j * _CS, _CS)]], rows_v, gsem
            ).wait()

            def fma_row(r, carry):
                for c in range(_D // _L):
                    sl = pl.ds(c * _L, _L)
                    rows_v[r, sl] = rows_v[r, sl] * _SCALE + pe_v[r, sl]
                return carry

            lax.fori_loop(0, _CS, fma_row, 0)
            pltpu.sync_copy(rows_v, out_hbm.at[b, pl.ds(s0, _CS)])


@jax.jit
def kernel(x, embedding_table, positional_encoding):
    run = pl.kernel(
        _body,
        out_type=jax.ShapeDtypeStruct((_B, _S, _D), jnp.float32),
        mesh=plsc.VectorSubcoreMesh(core_axis_name="c", subcore_axis_name="s"),
        scratch_types=[
            pltpu.VMEM((_B, _SEQ_PER_W), jnp.int32),
            pltpu.VMEM((_CS, _D), jnp.float32),
            pltpu.VMEM((_CS, _D), jnp.float32),
            pltpu.SemaphoreType.DMA,
        ],
    )
    return run(x, embedding_table, positional_encoding)


# speedup vs baseline: 1.2874x; 1.2874x over previous
import functools

import jax
import jax.numpy as jnp
from jax import lax
from jax.experimental import pallas as pl
from jax.experimental.pallas import tpu as pltpu
from jax.experimental.pallas import tpu_sc as plsc

_NC, _NS = 2, 16
_NW = _NC * _NS
_B, _S, _D = 4, 4096, 1024
_SEQ_PER_W = _S // _NW
_CS = 32
_NJ = _SEQ_PER_W // _CS
_L = 16
_SCALE = 32.0


def _body(x_hbm, table_hbm, pe_hbm, out_hbm,
          idx_v, pe_v, rows0, rows1, g0, g1, o0, o1, psem):
    wid = lax.axis_index("s") * _NC + lax.axis_index("c")
    s_base = wid * _SEQ_PER_W
    rows = (rows0, rows1)
    gsem = (g0, g1)
    osem = (o0, o1)

    pltpu.sync_copy(x_hbm.at[:, pl.ds(s_base, _SEQ_PER_W)], idx_v)

    tasks = [(j, b) for j in range(_NJ) for b in range(_B)]
    T = len(tasks)

    def start_gather(t, p):
        j, b = tasks[t]
        return pltpu.async_copy(
            table_hbm.at[idx_v.at[b, pl.ds(j * _CS, _CS)]], rows[p], gsem[p]
        )

    pe_cp = pltpu.async_copy(pe_hbm.at[pl.ds(s_base, _CS)], pe_v, psem)
    g_cp = {0: start_gather(0, 0)}
    o_cp = {}

    for t, (j, b) in enumerate(tasks):
        p = t % 2
        if t + 1 < T:
            if t - 1 >= 0:
                o_cp[t - 1].wait()
            g_cp[t + 1] = start_gather(t + 1, p ^ 1)
        if b == 0 and pe_cp is not None:
            pe_cp.wait()
            pe_cp = None
        g_cp[t].wait()

        def fma_row(r, carry):
            for c in range(_D // _L):
                sl = pl.ds(c * _L, _L)
                rows[p][r, sl] = rows[p][r, sl] * _SCALE + pe_v[r, sl]
            return carry

        lax.fori_loop(0, _CS, fma_row, 0)

        s0 = s_base + j * _CS
        o_cp[t] = pltpu.async_copy(rows[p], out_hbm.at[b, pl.ds(s0, _CS)],
                                   osem[p])
        if b == _B - 1 and j + 1 < _NJ:
            pe_cp = pltpu.async_copy(
                pe_hbm.at[pl.ds(s_base + (j + 1) * _CS, _CS)], pe_v, psem)

    o_cp[T - 2].wait()
    o_cp[T - 1].wait()


@jax.jit
def kernel(x, embedding_table, positional_encoding):
    run = pl.kernel(
        _body,
        out_type=jax.ShapeDtypeStruct((_B, _S, _D), jnp.float32),
        mesh=plsc.VectorSubcoreMesh(core_axis_name="c", subcore_axis_name="s"),
        scratch_types=[
            pltpu.VMEM((_B, _SEQ_PER_W), jnp.int32),
            pltpu.VMEM((_CS, _D), jnp.float32),
            pltpu.VMEM((_CS, _D), jnp.float32),
            pltpu.VMEM((_CS, _D), jnp.float32),
            pltpu.SemaphoreType.DMA,
            pltpu.SemaphoreType.DMA,
            pltpu.SemaphoreType.DMA,
            pltpu.SemaphoreType.DMA,
            pltpu.SemaphoreType.DMA,
        ],
    )
    return run(x, embedding_table, positional_encoding)


# speedup vs baseline: 1.8547x; 1.4407x over previous
import functools

import jax
import jax.numpy as jnp
from jax import lax
from jax.experimental import pallas as pl
from jax.experimental.pallas import tpu as pltpu
from jax.experimental.pallas import tpu_sc as plsc

_NC, _NS = 2, 16
_NW = _NC * _NS
_B, _S, _D = 4, 4096, 1024
_SEQ_PER_W = _S // _NW
_CS = 8
_NJ = _SEQ_PER_W // _CS
_R = _B * _CS
_L = 16
_CPR = _D // _L
_SCALE = 32.0
_UNROLL = 8


def _body(xc_hbm, table_hbm, pe_hbm, out_hbm,
          idx_v, pe0, pe1, rows0, rows1, g0, g1, p0, p1, o0, o1):
    wid = lax.axis_index("s") * _NC + lax.axis_index("c")
    s_base = wid * _SEQ_PER_W
    rows = (rows0, rows1)
    pe = (pe0, pe1)
    gsem = (g0, g1)
    psem = (p0, p1)
    osem = (o0, o1)

    pltpu.sync_copy(xc_hbm.at[wid], idx_v)

    def start_gather(t, p):
        return pltpu.async_copy(table_hbm.at[idx_v.at[t]], rows[p], gsem[p])

    def start_pe(t, p):
        return pltpu.async_copy(
            pe_hbm.at[pl.ds(s_base + t * _CS, _CS)], pe[p], psem[p])

    g_cp = {0: start_gather(0, 0)}
    pe_cp = {0: start_pe(0, 0)}
    o_cp = {}

    for t in range(_NJ):
        p = t & 1
        if t + 1 < _NJ:
            if t - 1 >= 0:
                for cp in o_cp[t - 1]:
                    cp.wait()
            g_cp[t + 1] = start_gather(t + 1, p ^ 1)
            pe_cp[t + 1] = start_pe(t + 1, p ^ 1)
        pe_cp[t].wait()
        g_cp[t].wait()

        rows_p, pe_p = rows[p], pe[p]

        @plsc.parallel_loop(0, _CS * _CPR, unroll=_UNROLL)
        def _fma(e):
            r = e // _CPR
            sl = pl.ds((e % _CPR) * _L, _L)
            pe_reg = pe_p[r, sl]
            for b in range(_B):
                rows_p[b * _CS + r, sl] = rows_p[b * _CS + r, sl] * _SCALE + pe_reg

        s0 = s_base + t * _CS
        o_cp[t] = [
            pltpu.async_copy(rows_p.at[pl.ds(b * _CS, _CS)],
                             out_hbm.at[b, pl.ds(s0, _CS)], osem[p])
            for b in range(_B)
        ]

    for t in (_NJ - 2, _NJ - 1):
        for cp in o_cp[t]:
            cp.wait()


@jax.jit
def kernel(x, embedding_table, positional_encoding):
    xc = (x.astype(jnp.int32)
          .reshape(_B, _NW, _NJ, _CS)
          .transpose(1, 2, 0, 3)
          .reshape(_NW, _NJ, _R))
    run = pl.kernel(
        _body,
        out_type=jax.ShapeDtypeStruct((_B, _S, _D), jnp.float32),
        mesh=plsc.VectorSubcoreMesh(core_axis_name="c", subcore_axis_name="s"),
        scratch_types=[
            pltpu.VMEM((_NJ, _R), jnp.int32),
            pltpu.VMEM((_CS, _D), jnp.float32),
            pltpu.VMEM((_CS, _D), jnp.float32),
            pltpu.VMEM((_R, _D), jnp.float32),
            pltpu.VMEM((_R, _D), jnp.float32),
            pltpu.SemaphoreType.DMA,
            pltpu.SemaphoreType.DMA,
            pltpu.SemaphoreType.DMA,
            pltpu.SemaphoreType.DMA,
            pltpu.SemaphoreType.DMA,
            pltpu.SemaphoreType.DMA,
        ],
    )
    return run(xc, embedding_table, positional_encoding)


# speedup vs baseline: 1.8850x; 1.0163x over previous
import functools

import jax
import jax.numpy as jnp
from jax import lax
from jax.experimental import pallas as pl
from jax.experimental.pallas import tpu as pltpu
from jax.experimental.pallas import tpu_sc as plsc

_NC, _NS = 2, 16
_NW = _NC * _NS
_B, _S, _D = 4, 4096, 1024
_SEQ_PER_W = _S // _NW
_CS = 8
_NJ = _SEQ_PER_W // _CS
_R = _B * _CS
_L = 16
_CPR = _D // _L
_SCALE = 32.0
_UNROLL = 8


def _body(xc_hbm, table_hbm, pe_hbm, out_hbm,
          idx_v, pe0, pe1, rows0, rows1, rows2, g0, g1, g2, p0, p1,
          o0, o1, o2):
    wid = lax.axis_index("s") * _NC + lax.axis_index("c")
    s_base = wid * _SEQ_PER_W
    rows = (rows0, rows1, rows2)
    pe = (pe0, pe1)
    gsem = (g0, g1, g2)
    psem = (p0, p1)
    osem = (o0, o1, o2)

    pltpu.sync_copy(xc_hbm.at[wid], idx_v)

    def start_gather(t):
        p = t % 3
        return pltpu.async_copy(table_hbm.at[idx_v.at[t]], rows[p], gsem[p])

    def start_pe(t):
        return pltpu.async_copy(
            pe_hbm.at[pl.ds(s_base + t * _CS, _CS)], pe[t % 2], psem[t % 2])

    g_cp = {0: start_gather(0), 1: start_gather(1)}
    pe_cp = {0: start_pe(0), 1: start_pe(1)}
    o_cp = {}

    for t in range(_NJ):
        p = t % 3
        if t + 2 < _NJ:
            if t - 1 >= 0:
                for cp in o_cp[t - 1]:
                    cp.wait()
            g_cp[t + 2] = start_gather(t + 2)
        pe_cp[t].wait()
        g_cp[t].wait()

        rows_p, pe_p = rows[p], pe[t % 2]

        @plsc.parallel_loop(0, _CS * _CPR, unroll=_UNROLL)
        def _fma(e):
            r = e // _CPR
            sl = pl.ds((e % _CPR) * _L, _L)
            pe_reg = pe_p[r, sl]
            for b in range(_B):
                rows_p[b * _CS + r, sl] = rows_p[b * _CS + r, sl] * _SCALE + pe_reg

        s0 = s_base + t * _CS
        o_cp[t] = [
            pltpu.async_copy(rows_p.at[pl.ds(b * _CS, _CS)],
                             out_hbm.at[b, pl.ds(s0, _CS)], osem[p])
            for b in range(_B)
        ]
        if t + 2 < _NJ:
            pe_cp[t + 2] = start_pe(t + 2)

    for t in (_NJ - 3, _NJ - 2, _NJ - 1):
        for cp in o_cp[t]:
            cp.wait()


@jax.jit
def kernel(x, embedding_table, positional_encoding):
    xc = (x.astype(jnp.int32)
          .reshape(_B, _NW, _NJ, _CS)
          .transpose(1, 2, 0, 3)
          .reshape(_NW, _NJ, _R))
    run = pl.kernel(
        _body,
        out_type=jax.ShapeDtypeStruct((_B, _S, _D), jnp.float32),
        mesh=plsc.VectorSubcoreMesh(core_axis_name="c", subcore_axis_name="s"),
        scratch_types=[
            pltpu.VMEM((_NJ, _R), jnp.int32),
            pltpu.VMEM((_CS, _D), jnp.float32),
            pltpu.VMEM((_CS, _D), jnp.float32),
            pltpu.VMEM((_R, _D), jnp.float32),
            pltpu.VMEM((_R, _D), jnp.float32),
            pltpu.VMEM((_R, _D), jnp.float32),
            pltpu.SemaphoreType.DMA,
            pltpu.SemaphoreType.DMA,
            pltpu.SemaphoreType.DMA,
            pltpu.SemaphoreType.DMA,
            pltpu.SemaphoreType.DMA,
            pltpu.SemaphoreType.DMA,
            pltpu.SemaphoreType.DMA,
            pltpu.SemaphoreType.DMA,
        ],
    )
    return run(xc, embedding_table, positional_encoding)
